# baseline (speedup 1.0000x reference)
import functools

import jax
import jax.numpy as jnp
from jax import lax
from jax.experimental import pallas as pl
from jax.experimental.pallas import tpu as pltpu
from jax.experimental.pallas import tpu_sc as plsc

NV = 20000
E = 320000
D = 128
ED = 16
NF = 2

NC = 2
NTILES = 16
CHUNK = 64
SENT_CH = E // CHUNK
EP_TOT = E + CHUNK
HALF = NV // NC
DUMP = HALF
ZROWS = 626
ACC_ROWS = ZROWS * NTILES
OUT_TILES = 10


def _make_edge_agg():
    mesh = plsc.VectorSubcoreMesh(core_axis_name="c", subcore_axis_name="s")

    nbuf = 2
    scratch = []
    for _ in range(nbuf):
        scratch += [
            pltpu.VMEM((CHUNK,), jnp.int32),
            pltpu.VMEM((CHUNK,), jnp.int32),
            pltpu.VMEM((CHUNK,), jnp.int32),
            pltpu.VMEM((CHUNK,), jnp.int32),
            pltpu.VMEM((CHUNK, D), jnp.float32),
            pltpu.VMEM((CHUNK, D), jnp.float32),
            pltpu.VMEM((CHUNK, D), jnp.float32),
        ]
    scratch.append(pltpu.VMEM((4, 16), jnp.int32))
    scratch.append(pltpu.VMEM_SHARED((ACC_ROWS, D), jnp.float32))
    scratch += [pltpu.SemaphoreType.DMA] * (6 * nbuf)

    @functools.partial(
        pl.kernel,
        mesh=mesh,
        out_type=jax.ShapeDtypeStruct((NC * HALF, D), jnp.float32),
        scratch_types=scratch,
    )
    def edge_agg(a_hbm, b_hbm, c_hbm, src_hbm, dst_hbm, cnt_hbm, out_hbm,
                 *rest):
        bufs = [rest[7 * i:7 * (i + 1)] for i in range(nbuf)]
        cnt_v = rest[7 * nbuf]
        acc_sh = rest[7 * nbuf + 1]
        sems = rest[7 * nbuf + 2:]
        s_src = sems[0:2]
        s_dst = sems[2:4]
        s_c = sems[4:6]
        s_a = sems[6:8]
        s_b = sems[8:10]
        s_scat = sems[10:12]
        src_v = [bufs[i][0] for i in range(nbuf)]
        dst_v = [bufs[i][1] for i in range(nbuf)]
        bidx_v = [bufs[i][2] for i in range(nbuf)]
        loc_v = [bufs[i][3] for i in range(nbuf)]
        a_v = [bufs[i][4] for i in range(nbuf)]
        b_v = [bufs[i][5] for i in range(nbuf)]
        c_v = [bufs[i][6] for i in range(nbuf)]

        cid = lax.axis_index("c")
        sid = lax.axis_index("s")
        base = cid * HALF

        zero16 = jnp.zeros((16,), jnp.float32)

        def zbody(i, carry):
            for j in range(D // 16):
                c_v[0][i, pl.ds(j * 16, 16)] = zero16
            return carry

        lax.fori_loop(0, CHUNK, zbody, 0)
        r0 = sid * ZROWS
        done = 0
        while done < ZROWS:
            sz = min(CHUNK, ZROWS - done)
            pltpu.sync_copy(c_v[0].at[pl.ds(0, sz)],
                            acc_sh.at[pl.ds(r0 + done, sz)])
            done += sz

        pltpu.sync_copy(cnt_hbm, cnt_v)
        nch_c = cnt_v[cid, pl.ds(0, 16)][0]
        chbase = cnt_v[2 + cid, pl.ds(0, 16)][0]
        nch_t = jnp.maximum((nch_c - sid + NTILES - 1) // NTILES, 0)
        nch2 = jnp.maximum((nch_t + 1) // 2 * 2, 2)
        pairs_m1 = (nch2 - 2) // 2

        def ch_idx(i):
            return jnp.where(i < nch_t, chbase + sid + i * NTILES, SENT_CH)

        plsc.subcore_barrier()

        def fire(b, ch, drain):
            eoff = ch * CHUNK
            if drain:
                pltpu.make_async_copy(
                    c_v[b], acc_sh.at[loc_v[b]], s_scat[b]).wait()
            d_src = pltpu.async_copy(
                src_hbm.at[pl.ds(eoff, CHUNK)], src_v[b], s_src[b])
            d_dst = pltpu.async_copy(
                dst_hbm.at[pl.ds(eoff, CHUNK)], dst_v[b], s_dst[b])
            pltpu.async_copy(c_hbm.at[pl.ds(eoff, CHUNK)], c_v[b], s_c[b])
            d_src.wait()
            pltpu.async_copy(a_hbm.at[src_v[b]], a_v[b], s_a[b])
            d_dst.wait()

            def ibody(g, carry):
                dd = dst_v[b][pl.ds(g * 16, 16)]
                bidx_v[b][pl.ds(g * 16, 16)] = jnp.minimum(
                    jnp.maximum(dd, 0), NV - 1)
                dl = dd - base
                ok = (dl >= 0) & (dl < HALF)
                loc_v[b][pl.ds(g * 16, 16)] = jnp.where(ok, dl, DUMP)
                return carry

            lax.fori_loop(0, CHUNK // 16, ibody, 0)
            pltpu.async_copy(b_hbm.at[bidx_v[b]], b_v[b], s_b[b])

        def finish(b):
            pltpu.make_async_copy(a_hbm.at[src_v[b]], a_v[b], s_a[b]).wait()
            pltpu.make_async_copy(b_hbm.at[bidx_v[b]], b_v[b], s_b[b]).wait()
            pltpu.make_async_copy(
                c_hbm.at[pl.ds(0, CHUNK)], c_v[b], s_c[b]).wait()

            def mbody(e, carry):
                for j in range(D // 16):
                    s_ = pl.ds(j * 16, 16)
                    c_v[b][e, s_] = jnp.maximum(
                        a_v[b][e, s_] + b_v[b][e, s_] + c_v[b][e, s_], 0.0)
                return carry

            lax.fori_loop(0, CHUNK, mbody, 0)
            pltpu.async_copy(c_v[b], acc_sh.at[loc_v[b]], s_scat[b],
                             add=True)

        fire(0, ch_idx(0), False)
        fire(1, ch_idx(1), False)

        def pair_body(p, carry):
            finish(0)
            fire(0, ch_idx(2 * p + 2), True)
            finish(1)
            fire(1, ch_idx(2 * p + 3), True)
            return carry

        lax.fori_loop(0, pairs_m1, pair_body, 0)
        finish(0)
        finish(1)
        for b in range(nbuf):
            pltpu.make_async_copy(
                c_v[b], acc_sh.at[loc_v[b]], s_scat[b]).wait()
        plsc.subcore_barrier()

        @pl.when(sid < OUT_TILES)
        def _copy_out():
            rr = sid * (HALF // OUT_TILES)
            pltpu.sync_copy(
                acc_sh.at[pl.ds(rr, HALF // OUT_TILES)],
                out_hbm.at[pl.ds(cid * HALF + rr, HALF // OUT_TILES)])

    return edge_agg


_EDGE_AGG_CACHE = []


def _sc_agg(A, B, C, src, dst, cnt):
    if not _EDGE_AGG_CACHE:
        _EDGE_AGG_CACHE.append(_make_edge_agg())
    return _EDGE_AGG_CACHE[0](A, B, C, src, dst, cnt)


def _route(src, dst, ea):
    key = (dst >= HALF).astype(jnp.int32)
    c1 = jnp.cumsum(key)
    n0 = E - c1[-1]
    idx = jnp.arange(E, dtype=jnp.int32)
    c0 = idx + 1 - c1
    pos = jnp.where(key == 0, c0 - 1, n0 + c1 - 1)
    perm = jnp.zeros((E,), jnp.int32).at[pos].set(
        idx, mode='drop', unique_indices=True)
    srcp = jnp.concatenate([src[perm], jnp.zeros((CHUNK,), jnp.int32)])
    dstp = jnp.concatenate(
        [dst[perm], jnp.full((CHUNK,), 1 << 30, jnp.int32)])
    eap = ea[perm]
    nch0 = (n0 + CHUNK - 1) // CHUNK
    base1 = n0 // CHUNK
    nch1 = (E - base1 * CHUNK + CHUNK - 1) // CHUNK
    cnt = jnp.stack([
        jnp.full((16,), nch0, jnp.int32),
        jnp.full((16,), nch1, jnp.int32),
        jnp.full((16,), 0, jnp.int32),
        jnp.full((16,), base1, jnp.int32),
    ])
    return srcp, dstp, eap, cnt


def _lin_body(x_ref, w_ref, b_ref, o_ref, *, act, res):
    y = jnp.dot(x_ref[...], w_ref[...], preferred_element_type=jnp.float32)
    y = y + b_ref[...]
    if act:
        y = jnp.maximum(y, 0.0)
    o_ref[...] = y


def _lin_res_body(x_ref, w_ref, b_ref, r_ref, o_ref):
    y = jnp.dot(x_ref[...], w_ref[...], preferred_element_type=jnp.float32)
    y = jnp.maximum(y + b_ref[...], 0.0)
    o_ref[...] = r_ref[...] + y


def _tc_lin(x, W, b, act=False, res=None, block_rows=1000):
    n, kdim = x.shape
    mdim = W.shape[1]
    assert n % block_rows == 0
    grid = (n // block_rows,)
    b2 = b.reshape(1, mdim)
    in_specs = [
        pl.BlockSpec((block_rows, kdim), lambda i: (i, 0)),
        pl.BlockSpec((kdim, mdim), lambda i: (0, 0)),
        pl.BlockSpec((1, mdim), lambda i: (0, 0)),
    ]
    args = [x, W, b2]
    if res is not None:
        in_specs.append(pl.BlockSpec((block_rows, mdim), lambda i: (i, 0)))
        args.append(res)
        body = _lin_res_body
    else:
        body = functools.partial(_lin_body, act=act, res=None)
    return pl.pallas_call(
        body,
        grid=grid,
        in_specs=in_specs,
        out_specs=pl.BlockSpec((block_rows, mdim), lambda i: (i, 0)),
        out_shape=jax.ShapeDtypeStruct((n, mdim), jnp.float32),
    )(*args)


def kernel(V0, S0, edge_index, edge_attr, params, M, obs_matrix):
    p = params
    WmVS, bmVS = p['fVS_msg']
    WuVS, buVS = p['fVS_upd']
    WmSV, bmSV = p['fSV_msg']
    WuSV, buSV = p['fSV_upd']
    Wsd, bsd = p['dStodV']
    Wds, bds = p['dVtodS']

    src = edge_index[0].astype(jnp.int32)
    dst = edge_index[1].astype(jnp.int32)
    src_f, dst_f, ea_f, cnt_f = _route(src, dst, edge_attr)
    src_r, dst_r, ea_r, cnt_r = _route(dst, src, edge_attr)

    ep_c = ((EP_TOT + 2047) // 2048) * 2048
    zpad = jnp.zeros((ep_c - E, ED), jnp.float32)
    zb = jnp.zeros((D,), jnp.float32)
    C_VS = _tc_lin(jnp.concatenate([ea_f, zpad], axis=0), WmVS[2 * D:], zb,
                   block_rows=2048)
    C_SV = _tc_lin(jnp.concatenate([ea_r, zpad], axis=0), WmSV[2 * D:], zb,
                   block_rows=2048)

    V = _tc_lin(V0, *p['embed_V'])
    S = _tc_lin(S0, *p['embed_S'])

    for _ in range(NF):
        S_proj = _tc_lin(S, Wsd, bsd)
        A1 = _tc_lin(V, WmVS[:D], zb)
        B1 = _tc_lin(S_proj, WmVS[D:2 * D], bmVS)
        agg1 = _sc_agg(A1, B1, C_VS, src_f, dst_f, cnt_f)
        S_out = _tc_lin(agg1, WuVS, buVS, res=S_proj)
        S = _tc_lin(S_out, Wds, bds)
        V_proj = _tc_lin(V, Wds, bds)
        A2 = _tc_lin(S, WmSV[:D], zb)
        B2 = _tc_lin(V_proj, WmSV[D:2 * D], bmSV)
        agg2 = _sc_agg(A2, B2, C_SV, src_r, dst_r, cnt_r)
        V_out = _tc_lin(agg2, WuSV, buSV, res=V_proj)
        V = _tc_lin(V_out, Wsd, bsd)

    P = _tc_lin(V, *p['extract_V']).reshape(-1, 3, 4)
    X = _tc_lin(S, *p['extract_S'])
    X = jnp.concatenate([X, jnp.ones((X.shape[0], 1), X.dtype)], axis=1)
    return (P, X)


# speedup vs baseline: 1.0890x; 1.0890x over previous
import functools

import jax
import jax.numpy as jnp
from jax import lax
from jax.experimental import pallas as pl
from jax.experimental.pallas import tpu as pltpu
from jax.experimental.pallas import tpu_sc as plsc

NV = 20000
E = 320000
D = 128
ED = 16
NF = 2

NC = 2
NTILES = 16
CHUNK = 64
SENT_CH = E // CHUNK
EP_TOT = E + CHUNK
HALF = NV // NC
DUMP = HALF
ZROWS = 626
ACC_ROWS = ZROWS * NTILES
OUT_TILES = 10


def _make_edge_agg():
    mesh = plsc.VectorSubcoreMesh(core_axis_name="c", subcore_axis_name="s")

    nbuf = 2
    nper = 8
    scratch = []
    for _ in range(nbuf):
        scratch += [
            pltpu.VMEM((CHUNK,), jnp.int32),
            pltpu.VMEM((CHUNK,), jnp.int32),
            pltpu.VMEM((CHUNK,), jnp.int32),
            pltpu.VMEM((CHUNK,), jnp.int32),
            pltpu.VMEM((CHUNK,), jnp.int32),
            pltpu.VMEM((CHUNK, D), jnp.float32),
            pltpu.VMEM((CHUNK, D), jnp.float32),
            pltpu.VMEM((CHUNK, D), jnp.float32),
        ]
    scratch.append(pltpu.VMEM((4, 16), jnp.int32))
    scratch.append(pltpu.VMEM_SHARED((ACC_ROWS, D), jnp.float32))
    scratch += [pltpu.SemaphoreType.DMA] * (7 * nbuf)

    @functools.partial(
        pl.kernel,
        mesh=mesh,
        out_type=jax.ShapeDtypeStruct((NC * HALF, D), jnp.float32),
        scratch_types=scratch,
    )
    def edge_agg(a_hbm, b_hbm, c_hbm, src_hbm, dst_hbm, perm_hbm, cnt_hbm,
                 out_hbm, *rest):
        bufs = [rest[nper * i:nper * (i + 1)] for i in range(nbuf)]
        cnt_v = rest[nper * nbuf]
        acc_sh = rest[nper * nbuf + 1]
        sems = rest[nper * nbuf + 2:]
        s_p = sems[0:2]
        s_src = sems[2:4]
        s_dst = sems[4:6]
        s_c = sems[6:8]
        s_a = sems[8:10]
        s_b = sems[10:12]
        s_scat = sems[12:14]
        perm_v = [bufs[i][0] for i in range(nbuf)]
        src_v = [bufs[i][1] for i in range(nbuf)]
        dst_v = [bufs[i][2] for i in range(nbuf)]
        bidx_v = [bufs[i][3] for i in range(nbuf)]
        loc_v = [bufs[i][4] for i in range(nbuf)]
        a_v = [bufs[i][5] for i in range(nbuf)]
        b_v = [bufs[i][6] for i in range(nbuf)]
        c_v = [bufs[i][7] for i in range(nbuf)]

        cid = lax.axis_index("c")
        sid = lax.axis_index("s")
        base = cid * HALF

        zero16 = jnp.zeros((16,), jnp.float32)

        def zbody(i, carry):
            for j in range(D // 16):
                c_v[0][i, pl.ds(j * 16, 16)] = zero16
            return carry

        lax.fori_loop(0, CHUNK, zbody, 0)
        r0 = sid * ZROWS
        done = 0
        while done < ZROWS:
            sz = min(CHUNK, ZROWS - done)
            pltpu.sync_copy(c_v[0].at[pl.ds(0, sz)],
                            acc_sh.at[pl.ds(r0 + done, sz)])
            done += sz

        pltpu.sync_copy(cnt_hbm, cnt_v)
        nch_c = cnt_v[cid, pl.ds(0, 16)][0]
        chbase = cnt_v[2 + cid, pl.ds(0, 16)][0]
        nch_t = jnp.maximum((nch_c - sid + NTILES - 1) // NTILES, 0)
        nch2 = jnp.maximum((nch_t + 1) // 2 * 2, 2)
        pairs_m1 = (nch2 - 2) // 2

        def ch_idx(i):
            return jnp.where(i < nch_t, chbase + sid + i * NTILES, SENT_CH)

        plsc.subcore_barrier()

        def fire(b, ch, drain):
            eoff = ch * CHUNK
            if drain:
                pltpu.make_async_copy(
                    c_v[b], acc_sh.at[loc_v[b]], s_scat[b]).wait()
            d_p = pltpu.async_copy(
                perm_hbm.at[pl.ds(eoff, CHUNK)], perm_v[b], s_p[b])
            d_p.wait()
            pltpu.async_copy(c_hbm.at[perm_v[b]], c_v[b], s_c[b])
            d_src = pltpu.async_copy(
                src_hbm.at[perm_v[b]], src_v[b], s_src[b])
            d_dst = pltpu.async_copy(
                dst_hbm.at[perm_v[b]], dst_v[b], s_dst[b])
            d_src.wait()
            pltpu.async_copy(a_hbm.at[src_v[b]], a_v[b], s_a[b])
            d_dst.wait()

            def ibody(g, carry):
                dd = dst_v[b][pl.ds(g * 16, 16)]
                bidx_v[b][pl.ds(g * 16, 16)] = jnp.minimum(
                    jnp.maximum(dd, 0), NV - 1)
                dl = dd - base
                ok = (dl >= 0) & (dl < HALF)
                loc_v[b][pl.ds(g * 16, 16)] = jnp.where(ok, dl, DUMP)
                return carry

            lax.fori_loop(0, CHUNK // 16, ibody, 0)
            pltpu.async_copy(b_hbm.at[bidx_v[b]], b_v[b], s_b[b])

        def finish(b):
            pltpu.make_async_copy(a_hbm.at[src_v[b]], a_v[b], s_a[b]).wait()
            pltpu.make_async_copy(b_hbm.at[bidx_v[b]], b_v[b], s_b[b]).wait()
            pltpu.make_async_copy(c_hbm.at[perm_v[b]], c_v[b], s_c[b]).wait()

            def mbody(e, carry):
                for j in range(D // 16):
                    s_ = pl.ds(j * 16, 16)
                    c_v[b][e, s_] = jnp.maximum(
                        a_v[b][e, s_] + b_v[b][e, s_] + c_v[b][e, s_], 0.0)
                return carry

            lax.fori_loop(0, CHUNK, mbody, 0)
            pltpu.async_copy(c_v[b], acc_sh.at[loc_v[b]], s_scat[b],
                             add=True)

        fire(0, ch_idx(0), False)
        fire(1, ch_idx(1), False)

        def pair_body(p, carry):
            finish(0)
            fire(0, ch_idx(2 * p + 2), True)
            finish(1)
            fire(1, ch_idx(2 * p + 3), True)
            return carry

        lax.fori_loop(0, pairs_m1, pair_body, 0)
        finish(0)
        finish(1)
        for b in range(nbuf):
            pltpu.make_async_copy(
                c_v[b], acc_sh.at[loc_v[b]], s_scat[b]).wait()
        plsc.subcore_barrier()

        @pl.when(sid < OUT_TILES)
        def _copy_out():
            rr = sid * (HALF // OUT_TILES)
            pltpu.sync_copy(
                acc_sh.at[pl.ds(rr, HALF // OUT_TILES)],
                out_hbm.at[pl.ds(cid * HALF + rr, HALF // OUT_TILES)])

    return edge_agg


_EDGE_AGG_CACHE = []


def _sc_agg(A, B, C, src, dst, perm, cnt):
    if not _EDGE_AGG_CACHE:
        _EDGE_AGG_CACHE.append(_make_edge_agg())
    return _EDGE_AGG_CACHE[0](A, B, C, src, dst, perm, cnt)


def _route(dst):
    key = (dst >= HALF).astype(jnp.int32)
    c1 = jnp.cumsum(key)
    n0 = E - c1[-1]
    idx = jnp.arange(E, dtype=jnp.int32)
    pos = jnp.where(key == 0, idx - c1, n0 + c1 - 1)
    perm = jnp.full((EP_TOT,), E, jnp.int32).at[pos].set(
        idx, mode='drop', unique_indices=True)
    nch0 = (n0 + CHUNK - 1) // CHUNK
    base1 = n0 // CHUNK
    nch1 = (E - base1 * CHUNK + CHUNK - 1) // CHUNK
    cnt = jnp.stack([
        jnp.full((16,), nch0, jnp.int32),
        jnp.full((16,), nch1, jnp.int32),
        jnp.full((16,), 0, jnp.int32),
        jnp.full((16,), base1, jnp.int32),
    ])
    return perm, cnt


def _lin_body(x_ref, w_ref, b_ref, o_ref, *, act):
    y = jnp.dot(x_ref[...], w_ref[...], preferred_element_type=jnp.float32)
    y = y + b_ref[...]
    if act:
        y = jnp.maximum(y, 0.0)
    o_ref[...] = y


def _lin_res_body(x_ref, w_ref, b_ref, r_ref, o_ref):
    y = jnp.dot(x_ref[...], w_ref[...], preferred_element_type=jnp.float32)
    y = jnp.maximum(y + b_ref[...], 0.0)
    o_ref[...] = r_ref[...] + y


def _tc_lin(x, W, b, act=False, res=None, block_rows=1000):
    n, kdim = x.shape
    mdim = W.shape[1]
    assert n % block_rows == 0
    grid = (n // block_rows,)
    b2 = b.reshape(1, mdim)
    in_specs = [
        pl.BlockSpec((block_rows, kdim), lambda i: (i, 0)),
        pl.BlockSpec((kdim, mdim), lambda i: (0, 0)),
        pl.BlockSpec((1, mdim), lambda i: (0, 0)),
    ]
    args = [x, W, b2]
    if res is not None:
        in_specs.append(pl.BlockSpec((block_rows, mdim), lambda i: (i, 0)))
        args.append(res)
        body = _lin_res_body
    else:
        body = functools.partial(_lin_body, act=act)
    return pl.pallas_call(
        body,
        grid=grid,
        in_specs=in_specs,
        out_specs=pl.BlockSpec((block_rows, mdim), lambda i: (i, 0)),
        out_shape=jax.ShapeDtypeStruct((n, mdim), jnp.float32),
    )(*args)


def kernel(V0, S0, edge_index, edge_attr, params, M, obs_matrix):
    p = params
    WmVS, bmVS = p['fVS_msg']
    WuVS, buVS = p['fVS_upd']
    WmSV, bmSV = p['fSV_msg']
    WuSV, buSV = p['fSV_upd']
    Wsd, bsd = p['dStodV']
    Wds, bds = p['dVtodS']

    src = edge_index[0].astype(jnp.int32)
    dst = edge_index[1].astype(jnp.int32)
    perm_f, cnt_f = _route(dst)
    perm_r, cnt_r = _route(src)
    zpad8 = jnp.zeros((8,), jnp.int32)
    bpad8 = jnp.full((8,), 1 << 30, jnp.int32)
    src_f = jnp.concatenate([src, zpad8])
    dst_f = jnp.concatenate([dst, bpad8])
    src_r = jnp.concatenate([dst, zpad8])
    dst_r = jnp.concatenate([src, bpad8])

    ep_c = ((E + 1 + 2047) // 2048) * 2048
    zpad = jnp.zeros((ep_c - E, ED), jnp.float32)
    ea_p = jnp.concatenate([edge_attr, zpad], axis=0)
    zb = jnp.zeros((D,), jnp.float32)
    C_VS = _tc_lin(ea_p, WmVS[2 * D:], zb, block_rows=2048)
    C_SV = _tc_lin(ea_p, WmSV[2 * D:], zb, block_rows=2048)

    V = _tc_lin(V0, *p['embed_V'])
    S = _tc_lin(S0, *p['embed_S'])

    for _ in range(NF):
        S_proj = _tc_lin(S, Wsd, bsd)
        A1 = _tc_lin(V, WmVS[:D], zb)
        B1 = _tc_lin(S_proj, WmVS[D:2 * D], bmVS)
        agg1 = _sc_agg(A1, B1, C_VS, src_f, dst_f, perm_f, cnt_f)
        S_out = _tc_lin(agg1, WuVS, buVS, res=S_proj)
        S = _tc_lin(S_out, Wds, bds)
        V_proj = _tc_lin(V, Wds, bds)
        A2 = _tc_lin(S, WmSV[:D], zb)
        B2 = _tc_lin(V_proj, WmSV[D:2 * D], bmSV)
        agg2 = _sc_agg(A2, B2, C_SV, src_r, dst_r, perm_r, cnt_r)
        V_out = _tc_lin(agg2, WuSV, buSV, res=V_proj)
        V = _tc_lin(V_out, Wsd, bsd)

    P = _tc_lin(V, *p['extract_V']).reshape(-1, 3, 4)
    X = _tc_lin(S, *p['extract_S'])
    X = jnp.concatenate([X, jnp.ones((X.shape[0], 1), X.dtype)], axis=1)
    return (P, X)


# speedup vs baseline: 1.1048x; 1.0145x over previous
import functools

import jax
import jax.numpy as jnp
from jax import lax
from jax.experimental import pallas as pl
from jax.experimental.pallas import tpu as pltpu
from jax.experimental.pallas import tpu_sc as plsc

NV = 20000
E = 320000
D = 128
ED = 16
NF = 2

NC = 2
NTILES = 16
CHUNK = 64
SENT_CH = E // CHUNK
EP_TOT = E + CHUNK
HALF = NV // NC
DUMP = HALF
ZROWS = 626
ACC_ROWS = ZROWS * NTILES
OUT_TILES = 10


def _make_edge_agg():
    mesh = plsc.VectorSubcoreMesh(core_axis_name="c", subcore_axis_name="s")

    nbuf = 2
    nper = 8
    scratch = []
    for _ in range(nbuf):
        scratch += [
            pltpu.VMEM((CHUNK,), jnp.int32),
            pltpu.VMEM((CHUNK,), jnp.int32),
            pltpu.VMEM((CHUNK,), jnp.int32),
            pltpu.VMEM((CHUNK,), jnp.int32),
            pltpu.VMEM((CHUNK,), jnp.int32),
            pltpu.VMEM((CHUNK, D), jnp.float32),
            pltpu.VMEM((CHUNK, D), jnp.float32),
            pltpu.VMEM((CHUNK, D), jnp.float32),
        ]
    scratch.append(pltpu.VMEM((4, 16), jnp.int32))
    scratch.append(pltpu.VMEM_SHARED((ACC_ROWS, D), jnp.float32))
    scratch += [pltpu.SemaphoreType.DMA] * (7 * nbuf)

    @functools.partial(
        pl.kernel,
        mesh=mesh,
        out_type=jax.ShapeDtypeStruct((NC * HALF, D), jnp.float32),
        scratch_types=scratch,
    )
    def edge_agg(a_hbm, b_hbm, c_hbm, src_hbm, dst_hbm, perm_hbm, cnt_hbm,
                 out_hbm, *rest):
        bufs = [rest[nper * i:nper * (i + 1)] for i in range(nbuf)]
        cnt_v = rest[nper * nbuf]
        acc_sh = rest[nper * nbuf + 1]
        sems = rest[nper * nbuf + 2:]
        s_p = sems[0:2]
        s_src = sems[2:4]
        s_dst = sems[4:6]
        s_c = sems[6:8]
        s_a = sems[8:10]
        s_b = sems[10:12]
        s_scat = sems[12:14]
        perm_v = [bufs[i][0] for i in range(nbuf)]
        src_v = [bufs[i][1] for i in range(nbuf)]
        dst_v = [bufs[i][2] for i in range(nbuf)]
        bidx_v = [bufs[i][3] for i in range(nbuf)]
        loc_v = [bufs[i][4] for i in range(nbuf)]
        a_v = [bufs[i][5] for i in range(nbuf)]
        b_v = [bufs[i][6] for i in range(nbuf)]
        c_v = [bufs[i][7] for i in range(nbuf)]

        cid = lax.axis_index("c")
        sid = lax.axis_index("s")
        base = cid * HALF

        zero16 = jnp.zeros((16,), jnp.float32)

        def zbody(i, carry):
            for j in range(D // 16):
                c_v[0][i, pl.ds(j * 16, 16)] = zero16
            return carry

        lax.fori_loop(0, CHUNK, zbody, 0)
        r0 = sid * ZROWS
        done = 0
        while done < ZROWS:
            sz = min(CHUNK, ZROWS - done)
            pltpu.sync_copy(c_v[0].at[pl.ds(0, sz)],
                            acc_sh.at[pl.ds(r0 + done, sz)])
            done += sz

        pltpu.sync_copy(cnt_hbm, cnt_v)
        nch_c = cnt_v[cid, pl.ds(0, 16)][0]
        chbase = cnt_v[2 + cid, pl.ds(0, 16)][0]
        nch_t = jnp.maximum((nch_c - sid + NTILES - 1) // NTILES, 0)
        nch2 = jnp.maximum((nch_t + 1) // 2 * 2, 2)
        pairs_m1 = (nch2 - 2) // 2

        def ch_idx(i):
            return jnp.where(i < nch_t, chbase + sid + i * NTILES, SENT_CH)

        plsc.subcore_barrier()

        def fire(b, ch, drain):
            eoff = ch * CHUNK
            if drain:
                pltpu.make_async_copy(
                    c_v[b], acc_sh.at[loc_v[b]], s_scat[b]).wait()
            d_p = pltpu.async_copy(
                perm_hbm.at[pl.ds(eoff, CHUNK)], perm_v[b], s_p[b])
            d_src = pltpu.async_copy(
                src_hbm.at[pl.ds(eoff, CHUNK)], src_v[b], s_src[b])
            d_dst = pltpu.async_copy(
                dst_hbm.at[pl.ds(eoff, CHUNK)], dst_v[b], s_dst[b])
            d_p.wait()
            pltpu.async_copy(c_hbm.at[perm_v[b]], c_v[b], s_c[b])
            d_src.wait()
            pltpu.async_copy(a_hbm.at[src_v[b]], a_v[b], s_a[b])
            d_dst.wait()

            def ibody(g, carry):
                dd = dst_v[b][pl.ds(g * 16, 16)]
                bidx_v[b][pl.ds(g * 16, 16)] = jnp.minimum(
                    jnp.maximum(dd, 0), NV - 1)
                dl = dd - base
                ok = (dl >= 0) & (dl < HALF)
                loc_v[b][pl.ds(g * 16, 16)] = jnp.where(ok, dl, DUMP)
                return carry

            lax.fori_loop(0, CHUNK // 16, ibody, 0)
            pltpu.async_copy(b_hbm.at[bidx_v[b]], b_v[b], s_b[b])

        def finish(b):
            pltpu.make_async_copy(a_hbm.at[src_v[b]], a_v[b], s_a[b]).wait()
            pltpu.make_async_copy(b_hbm.at[bidx_v[b]], b_v[b], s_b[b]).wait()
            pltpu.make_async_copy(c_hbm.at[perm_v[b]], c_v[b], s_c[b]).wait()

            def mbody(e, carry):
                for j in range(D // 16):
                    s_ = pl.ds(j * 16, 16)
                    c_v[b][e, s_] = jnp.maximum(
                        a_v[b][e, s_] + b_v[b][e, s_] + c_v[b][e, s_], 0.0)
                return carry

            lax.fori_loop(0, CHUNK, mbody, 0)
            pltpu.async_copy(c_v[b], acc_sh.at[loc_v[b]], s_scat[b],
                             add=True)

        fire(0, ch_idx(0), False)
        fire(1, ch_idx(1), False)

        def pair_body(p, carry):
            finish(0)
            fire(0, ch_idx(2 * p + 2), True)
            finish(1)
            fire(1, ch_idx(2 * p + 3), True)
            return carry

        lax.fori_loop(0, pairs_m1, pair_body, 0)
        finish(0)
        finish(1)
        for b in range(nbuf):
            pltpu.make_async_copy(
                c_v[b], acc_sh.at[loc_v[b]], s_scat[b]).wait()
        plsc.subcore_barrier()

        @pl.when(sid < OUT_TILES)
        def _copy_out():
            rr = sid * (HALF // OUT_TILES)
            pltpu.sync_copy(
                acc_sh.at[pl.ds(rr, HALF // OUT_TILES)],
                out_hbm.at[pl.ds(cid * HALF + rr, HALF // OUT_TILES)])

    return edge_agg


_EDGE_AGG_CACHE = []


def _sc_agg(A, B, C, src, dst, perm, cnt):
    if not _EDGE_AGG_CACHE:
        _EDGE_AGG_CACHE.append(_make_edge_agg())
    return _EDGE_AGG_CACHE[0](A, B, C, src, dst, perm, cnt)


def _route(dst):
    key = (dst >= HALF).astype(jnp.int32)
    c1 = jnp.cumsum(key)
    n0 = E - c1[-1]
    idx = jnp.arange(E, dtype=jnp.int32)
    pos = jnp.where(key == 0, idx - c1, n0 + c1 - 1)
    perm = jnp.full((EP_TOT,), E, jnp.int32).at[pos].set(
        idx, mode='drop', unique_indices=True)
    nch0 = (n0 + CHUNK - 1) // CHUNK
    base1 = n0 // CHUNK
    nch1 = (E - base1 * CHUNK + CHUNK - 1) // CHUNK
    cnt = jnp.stack([
        jnp.full((16,), nch0, jnp.int32),
        jnp.full((16,), nch1, jnp.int32),
        jnp.full((16,), 0, jnp.int32),
        jnp.full((16,), base1, jnp.int32),
    ])
    return perm, cnt


def _lin_body(x_ref, w_ref, b_ref, o_ref, *, act):
    y = jnp.dot(x_ref[...], w_ref[...], preferred_element_type=jnp.float32)
    y = y + b_ref[...]
    if act:
        y = jnp.maximum(y, 0.0)
    o_ref[...] = y


def _lin_res_body(x_ref, w_ref, b_ref, r_ref, o_ref):
    y = jnp.dot(x_ref[...], w_ref[...], preferred_element_type=jnp.float32)
    y = jnp.maximum(y + b_ref[...], 0.0)
    o_ref[...] = r_ref[...] + y


def _tc_lin(x, W, b, act=False, res=None, block_rows=1000):
    n, kdim = x.shape
    mdim = W.shape[1]
    assert n % block_rows == 0
    grid = (n // block_rows,)
    b2 = b.reshape(1, mdim)
    in_specs = [
        pl.BlockSpec((block_rows, kdim), lambda i: (i, 0)),
        pl.BlockSpec((kdim, mdim), lambda i: (0, 0)),
        pl.BlockSpec((1, mdim), lambda i: (0, 0)),
    ]
    args = [x, W, b2]
    if res is not None:
        in_specs.append(pl.BlockSpec((block_rows, mdim), lambda i: (i, 0)))
        args.append(res)
        body = _lin_res_body
    else:
        body = functools.partial(_lin_body, act=act)
    return pl.pallas_call(
        body,
        grid=grid,
        in_specs=in_specs,
        out_specs=pl.BlockSpec((block_rows, mdim), lambda i: (i, 0)),
        out_shape=jax.ShapeDtypeStruct((n, mdim), jnp.float32),
    )(*args)


def kernel(V0, S0, edge_index, edge_attr, params, M, obs_matrix):
    p = params
    WmVS, bmVS = p['fVS_msg']
    WuVS, buVS = p['fVS_upd']
    WmSV, bmSV = p['fSV_msg']
    WuSV, buSV = p['fSV_upd']
    Wsd, bsd = p['dStodV']
    Wds, bds = p['dVtodS']

    src = edge_index[0].astype(jnp.int32)
    dst = edge_index[1].astype(jnp.int32)
    perm_f, cnt_f = _route(dst)
    perm_r, cnt_r = _route(src)
    zpad8 = jnp.zeros((8,), jnp.int32)
    bpad8 = jnp.full((8,), 1 << 30, jnp.int32)
    src_z = jnp.concatenate([src, zpad8])
    dst_b = jnp.concatenate([dst, bpad8])
    dst_z = jnp.concatenate([dst, zpad8])
    src_b = jnp.concatenate([src, bpad8])
    srcp_f, dstp_f = src_z[perm_f], dst_b[perm_f]
    srcp_r, dstp_r = dst_z[perm_r], src_b[perm_r]

    ep_c = ((E + 1 + 2047) // 2048) * 2048
    zpad = jnp.zeros((ep_c - E, ED), jnp.float32)
    ea_p = jnp.concatenate([edge_attr, zpad], axis=0)
    zb = jnp.zeros((D,), jnp.float32)
    C_VS = _tc_lin(ea_p, WmVS[2 * D:], zb, block_rows=2048)
    C_SV = _tc_lin(ea_p, WmSV[2 * D:], zb, block_rows=2048)

    V = _tc_lin(V0, *p['embed_V'])
    S = _tc_lin(S0, *p['embed_S'])

    for _ in range(NF):
        S_proj = _tc_lin(S, Wsd, bsd)
        A1 = _tc_lin(V, WmVS[:D], zb)
        B1 = _tc_lin(S_proj, WmVS[D:2 * D], bmVS)
        agg1 = _sc_agg(A1, B1, C_VS, srcp_f, dstp_f, perm_f, cnt_f)
        S_out = _tc_lin(agg1, WuVS, buVS, res=S_proj)
        S = _tc_lin(S_out, Wds, bds)
        V_proj = _tc_lin(V, Wds, bds)
        A2 = _tc_lin(S, WmSV[:D], zb)
        B2 = _tc_lin(V_proj, WmSV[D:2 * D], bmSV)
        agg2 = _sc_agg(A2, B2, C_SV, srcp_r, dstp_r, perm_r, cnt_r)
        V_out = _tc_lin(agg2, WuSV, buSV, res=V_proj)
        V = _tc_lin(V_out, Wsd, bsd)

    P = _tc_lin(V, *p['extract_V']).reshape(-1, 3, 4)
    X = _tc_lin(S, *p['extract_S'])
    X = jnp.concatenate([X, jnp.ones((X.shape[0], 1), X.dtype)], axis=1)
    return (P, X)


# speedup vs baseline: 1.1254x; 1.0187x over previous
import functools

import jax
import jax.numpy as jnp
from jax import lax
from jax.experimental import pallas as pl
from jax.experimental.pallas import tpu as pltpu
from jax.experimental.pallas import tpu_sc as plsc

NV = 20000
E = 320000
D = 128
ED = 16
NF = 2

NC = 2
NTILES = 16
CHUNK = 64
SENT_CH = E // CHUNK
EP_TOT = E + CHUNK
HALF = NV // NC
DUMP = HALF
ZROWS = 626
ACC_ROWS = ZROWS * NTILES
OUT_TILES = 10


def _make_edge_agg():
    mesh = plsc.VectorSubcoreMesh(core_axis_name="c", subcore_axis_name="s")

    nbuf = 2
    nper = 8
    scratch = []
    for _ in range(nbuf):
        scratch += [
            pltpu.VMEM((CHUNK,), jnp.int32),
            pltpu.VMEM((CHUNK,), jnp.int32),
            pltpu.VMEM((CHUNK,), jnp.int32),
            pltpu.VMEM((CHUNK,), jnp.int32),
            pltpu.VMEM((CHUNK,), jnp.int32),
            pltpu.VMEM((CHUNK, D), jnp.float32),
            pltpu.VMEM((CHUNK, D), jnp.float32),
            pltpu.VMEM((CHUNK, D), jnp.float32),
        ]
    scratch.append(pltpu.VMEM((4, 16), jnp.int32))
    scratch.append(pltpu.VMEM_SHARED((ACC_ROWS, D), jnp.float32))
    scratch += [pltpu.SemaphoreType.DMA] * (7 * nbuf)

    @functools.partial(
        pl.kernel,
        mesh=mesh,
        out_type=jax.ShapeDtypeStruct((NC * HALF, D), jnp.float32),
        scratch_types=scratch,
    )
    def edge_agg(a_hbm, b_hbm, c_hbm, src_hbm, dst_hbm, perm_hbm, cnt_hbm,
                 out_hbm, *rest):
        bufs = [rest[nper * i:nper * (i + 1)] for i in range(nbuf)]
        cnt_v = rest[nper * nbuf]
        acc_sh = rest[nper * nbuf + 1]
        sems = rest[nper * nbuf + 2:]
        s_p = sems[0:2]
        s_src = sems[2:4]
        s_dst = sems[4:6]
        s_c = sems[6:8]
        s_a = sems[8:10]
        s_b = sems[10:12]
        s_scat = sems[12:14]
        perm_v = [bufs[i][0] for i in range(nbuf)]
        src_v = [bufs[i][1] for i in range(nbuf)]
        dst_v = [bufs[i][2] for i in range(nbuf)]
        bidx_v = [bufs[i][3] for i in range(nbuf)]
        loc_v = [bufs[i][4] for i in range(nbuf)]
        a_v = [bufs[i][5] for i in range(nbuf)]
        b_v = [bufs[i][6] for i in range(nbuf)]
        c_v = [bufs[i][7] for i in range(nbuf)]

        cid = lax.axis_index("c")
        sid = lax.axis_index("s")
        base = cid * HALF

        zero16 = jnp.zeros((16,), jnp.float32)

        def zbody(i, carry):
            for j in range(D // 16):
                c_v[0][i, pl.ds(j * 16, 16)] = zero16
            return carry

        lax.fori_loop(0, CHUNK, zbody, 0)
        r0 = sid * ZROWS
        done = 0
        while done < ZROWS:
            sz = min(CHUNK, ZROWS - done)
            pltpu.sync_copy(c_v[0].at[pl.ds(0, sz)],
                            acc_sh.at[pl.ds(r0 + done, sz)])
            done += sz

        pltpu.sync_copy(cnt_hbm, cnt_v)
        nch_c = cnt_v[cid, pl.ds(0, 16)][0]
        chbase = cnt_v[2 + cid, pl.ds(0, 16)][0]
        nch_t = jnp.maximum((nch_c - sid + NTILES - 1) // NTILES, 0)
        nch2 = jnp.maximum((nch_t + 1) // 2 * 2, 2)
        pairs_m1 = (nch2 - 2) // 2

        def ch_idx(i):
            return jnp.where(i < nch_t, chbase + sid + i * NTILES, SENT_CH)

        plsc.subcore_barrier()

        def fire(b, ch, drain):
            eoff = ch * CHUNK
            if drain:
                pltpu.make_async_copy(
                    c_v[b], acc_sh.at[loc_v[b]], s_scat[b]).wait()
            d_p = pltpu.async_copy(
                perm_hbm.at[pl.ds(eoff, CHUNK)], perm_v[b], s_p[b])
            d_src = pltpu.async_copy(
                src_hbm.at[pl.ds(eoff, CHUNK)], src_v[b], s_src[b])
            d_dst = pltpu.async_copy(
                dst_hbm.at[pl.ds(eoff, CHUNK)], dst_v[b], s_dst[b])
            d_p.wait()
            pltpu.async_copy(c_hbm.at[perm_v[b]], c_v[b], s_c[b])
            d_src.wait()
            pltpu.async_copy(a_hbm.at[src_v[b]], a_v[b], s_a[b])
            d_dst.wait()

            def ibody(g, carry):
                dd = dst_v[b][pl.ds(g * 16, 16)]
                pp = perm_v[b][pl.ds(g * 16, 16)]
                bidx_v[b][pl.ds(g * 16, 16)] = jnp.minimum(
                    jnp.maximum(dd, 0), NV - 1)
                dl = dd - base
                ok = (dl >= 0) & (dl < HALF) & (pp < E)
                loc_v[b][pl.ds(g * 16, 16)] = jnp.where(ok, dl, DUMP)
                return carry

            lax.fori_loop(0, CHUNK // 16, ibody, 0)
            pltpu.async_copy(b_hbm.at[bidx_v[b]], b_v[b], s_b[b])

        def finish(b):
            pltpu.make_async_copy(a_hbm.at[src_v[b]], a_v[b], s_a[b]).wait()
            pltpu.make_async_copy(b_hbm.at[bidx_v[b]], b_v[b], s_b[b]).wait()
            pltpu.make_async_copy(c_hbm.at[perm_v[b]], c_v[b], s_c[b]).wait()

            def mbody(e, carry):
                for j in range(D // 16):
                    s_ = pl.ds(j * 16, 16)
                    c_v[b][e, s_] = jnp.maximum(
                        a_v[b][e, s_] + b_v[b][e, s_] + c_v[b][e, s_], 0.0)
                return carry

            lax.fori_loop(0, CHUNK, mbody, 0)
            pltpu.async_copy(c_v[b], acc_sh.at[loc_v[b]], s_scat[b],
                             add=True)

        fire(0, ch_idx(0), False)
        fire(1, ch_idx(1), False)

        def pair_body(p, carry):
            finish(0)
            fire(0, ch_idx(2 * p + 2), True)
            finish(1)
            fire(1, ch_idx(2 * p + 3), True)
            return carry

        lax.fori_loop(0, pairs_m1, pair_body, 0)
        finish(0)
        finish(1)
        for b in range(nbuf):
            pltpu.make_async_copy(
                c_v[b], acc_sh.at[loc_v[b]], s_scat[b]).wait()
        plsc.subcore_barrier()

        @pl.when(sid < OUT_TILES)
        def _copy_out():
            rr = sid * (HALF // OUT_TILES)
            pltpu.sync_copy(
                acc_sh.at[pl.ds(rr, HALF // OUT_TILES)],
                out_hbm.at[pl.ds(cid * HALF + rr, HALF // OUT_TILES)])

    return edge_agg


_EDGE_AGG_CACHE = []


def _sc_agg(A, B, C, src, dst, perm, cnt):
    if not _EDGE_AGG_CACHE:
        _EDGE_AGG_CACHE.append(_make_edge_agg())
    return _EDGE_AGG_CACHE[0](A, B, C, src, dst, perm, cnt)


def _route(dst):
    key = (dst >= HALF).astype(jnp.int32)
    c1 = jnp.cumsum(key)
    n0 = E - c1[-1]
    idx = jnp.arange(E, dtype=jnp.int32)
    pos = jnp.where(key == 0, idx - c1, n0 + c1 - 1)
    perm = jnp.full((EP_TOT,), E, jnp.int32).at[pos].set(
        idx, mode='drop', unique_indices=True)
    nch0 = (n0 + CHUNK - 1) // CHUNK
    base1 = n0 // CHUNK
    nch1 = (E - base1 * CHUNK + CHUNK - 1) // CHUNK
    cnt = jnp.stack([
        jnp.full((16,), nch0, jnp.int32),
        jnp.full((16,), nch1, jnp.int32),
        jnp.full((16,), 0, jnp.int32),
        jnp.full((16,), base1, jnp.int32),
    ])
    return perm, cnt


def _lin_body(x_ref, w_ref, b_ref, o_ref, *, act):
    y = jnp.dot(x_ref[...], w_ref[...], preferred_element_type=jnp.float32)
    y = y + b_ref[...]
    if act:
        y = jnp.maximum(y, 0.0)
    o_ref[...] = y


def _lin_res_body(x_ref, w_ref, b_ref, r_ref, o_ref):
    y = jnp.dot(x_ref[...], w_ref[...], preferred_element_type=jnp.float32)
    y = jnp.maximum(y + b_ref[...], 0.0)
    o_ref[...] = r_ref[...] + y


def _tc_lin(x, W, b, act=False, res=None, block_rows=1000):
    n, kdim = x.shape
    mdim = W.shape[1]
    assert n % block_rows == 0
    grid = (n // block_rows,)
    b2 = b.reshape(1, mdim)
    in_specs = [
        pl.BlockSpec((block_rows, kdim), lambda i: (i, 0)),
        pl.BlockSpec((kdim, mdim), lambda i: (0, 0)),
        pl.BlockSpec((1, mdim), lambda i: (0, 0)),
    ]
    args = [x, W, b2]
    if res is not None:
        in_specs.append(pl.BlockSpec((block_rows, mdim), lambda i: (i, 0)))
        args.append(res)
        body = _lin_res_body
    else:
        body = functools.partial(_lin_body, act=act)
    return pl.pallas_call(
        body,
        grid=grid,
        in_specs=in_specs,
        out_specs=pl.BlockSpec((block_rows, mdim), lambda i: (i, 0)),
        out_shape=jax.ShapeDtypeStruct((n, mdim), jnp.float32),
    )(*args)


def kernel(V0, S0, edge_index, edge_attr, params, M, obs_matrix):
    p = params
    WmVS, bmVS = p['fVS_msg']
    WuVS, buVS = p['fVS_upd']
    WmSV, bmSV = p['fSV_msg']
    WuSV, buSV = p['fSV_upd']
    Wsd, bsd = p['dStodV']
    Wds, bds = p['dVtodS']

    src = edge_index[0].astype(jnp.int32)
    dst = edge_index[1].astype(jnp.int32)
    perm_f, cnt_f = _route(dst)
    perm_r, cnt_r = _route(src)
    zpad8 = jnp.zeros((8,), jnp.int32)
    src_x = jnp.concatenate([src, zpad8])
    dst_x = jnp.concatenate([dst, zpad8])
    srcp_f, dstp_f = src_x[perm_f], dst_x[perm_f]
    srcp_r, dstp_r = dst_x[perm_r], src_x[perm_r]

    ep_c = ((E + 1 + 2047) // 2048) * 2048
    zpad = jnp.zeros((ep_c - E, ED), jnp.float32)
    ea_p = jnp.concatenate([edge_attr, zpad], axis=0)
    zb = jnp.zeros((D,), jnp.float32)
    C_VS = _tc_lin(ea_p, WmVS[2 * D:], zb, block_rows=2048)
    C_SV = _tc_lin(ea_p, WmSV[2 * D:], zb, block_rows=2048)

    V = _tc_lin(V0, *p['embed_V'])
    S = _tc_lin(S0, *p['embed_S'])

    for _ in range(NF):
        S_proj = _tc_lin(S, Wsd, bsd)
        A1 = _tc_lin(V, WmVS[:D], zb)
        B1 = _tc_lin(S_proj, WmVS[D:2 * D], bmVS)
        agg1 = _sc_agg(A1, B1, C_VS, srcp_f, dstp_f, perm_f, cnt_f)
        S_out = _tc_lin(agg1, WuVS, buVS, res=S_proj)
        S = _tc_lin(S_out, Wds, bds)
        V_proj = _tc_lin(V, Wds, bds)
        A2 = _tc_lin(S, WmSV[:D], zb)
        B2 = _tc_lin(V_proj, WmSV[D:2 * D], bmSV)
        agg2 = _sc_agg(A2, B2, C_SV, srcp_r, dstp_r, perm_r, cnt_r)
        V_out = _tc_lin(agg2, WuSV, buSV, res=V_proj)
        V = _tc_lin(V_out, Wsd, bsd)

    P = _tc_lin(V, *p['extract_V']).reshape(-1, 3, 4)
    X = _tc_lin(S, *p['extract_S'])
    X = jnp.concatenate([X, jnp.ones((X.shape[0], 1), X.dtype)], axis=1)
    return (P, X)
